# baseline (device time: 199992 ns/iter reference)
import jax
import jax.numpy as jnp
from jax import lax
from jax.experimental import pallas as pl
from jax.experimental.pallas import tpu as pltpu

N_DEV = 16
B, SQ, SKV, DH = 2, 512, 512, 64
HQ_LOCAL = 8
D_LOCAL = HQ_LOCAL * DH
D_MODEL = 768
ROWS = B * SQ
CHUNK = ROWS // N_DEV

_MESH = pl.DeviceIdType.MESH


def _body(x_ref, wq_ref, k_hbm, v_hbm, wo_ref, out_ref,
          k_scr, v_scr, snd_ref, ctx_ref, a2a_ref, red_ref, flat_ref,
          kv_sems, s1_send, s1_recv, s2_send, s2_recv):
    my = lax.axis_index("i")

    kv_copies = []
    for b in range(B):
        ck = pltpu.make_async_copy(
            k_hbm.at[b, :, my, :], k_scr.at[b], kv_sems.at[2 * b])
        cv = pltpu.make_async_copy(
            v_hbm.at[b, :, my, :], v_scr.at[b], kv_sems.at[2 * b + 1])
        ck.start()
        cv.start()
        kv_copies.append((ck, cv))

    barrier_sem = pltpu.get_barrier_semaphore()
    for dj in range(1, N_DEV):
        peer = lax.rem(my + dj, N_DEV)
        pl.semaphore_signal(barrier_sem, inc=1, device_id=(peer,),
                            device_id_type=_MESH)
    pl.semaphore_wait(barrier_sem, N_DEV - 1)

    qb_i = lax.broadcasted_iota(jnp.int32, (SQ, SKV), 0) // 64
    kb_i = lax.broadcasted_iota(jnp.int32, (SQ, SKV), 1) // 64
    bias = jnp.where((kb_i % 4) == (qb_i % 4), 0.0, -1e9)
    for b in range(B):
        q_b = jnp.dot(x_ref[b], wq_ref[...],
                      preferred_element_type=jnp.float32).astype(jnp.bfloat16)
        ck, cv = kv_copies[b]
        ck.wait()
        cv.wait()
        k_all = k_scr[b].astype(jnp.bfloat16)
        v_all = v_scr[b].astype(jnp.bfloat16)
        for h in range(HQ_LOCAL):
            q = q_b[:, h * DH:(h + 1) * DH]
            k = k_all[:, h * DH:(h + 1) * DH]
            s = lax.dot_general(q, k, (((1,), (1,)), ((), ())),
                                preferred_element_type=jnp.float32)
            w = jnp.exp(s * 0.125 + bias)
            w = w / jnp.sum(w, axis=1, keepdims=True)
            ctx = jnp.dot(w.astype(jnp.bfloat16), v_all[:, h * DH:(h + 1) * DH],
                          preferred_element_type=jnp.float32)
            ctx_ref[b, :, h * DH:(h + 1) * DH] = ctx.astype(jnp.bfloat16)
        proj = jnp.dot(ctx_ref[b], wo_ref[...],
                       preferred_element_type=jnp.float32)
        snd_ref[b * SQ:(b + 1) * SQ, :] = proj.astype(jnp.bfloat16)

    sends1 = []
    for dj in range(1, N_DEV):
        d = lax.rem(my + dj, N_DEV)
        rdma = pltpu.make_async_remote_copy(
            src_ref=snd_ref.at[pl.ds(d * CHUNK, CHUNK), :],
            dst_ref=a2a_ref.at[dj - 1],
            send_sem=s1_send.at[dj - 1],
            recv_sem=s1_recv.at[dj - 1],
            device_id=(d,),
            device_id_type=_MESH,
        )
        rdma.start()
        sends1.append(rdma)

    red = snd_ref[pl.ds(my * CHUNK, CHUNK), :].astype(jnp.float32)
    for k in range(N_DEV - 1):
        recv = pltpu.make_async_remote_copy(
            src_ref=a2a_ref.at[k], dst_ref=a2a_ref.at[k],
            send_sem=s1_send.at[k], recv_sem=s1_recv.at[k],
            device_id=(my,), device_id_type=_MESH,
        )
        recv.wait_recv()
        red = red + a2a_ref[k].astype(jnp.float32)
    red_ref[...] = red.astype(jnp.bfloat16)
    flat_ref[pl.ds(my * CHUNK, CHUNK), :] = red_ref[...]
    for r in sends1:
        r.wait_send()

    sends2 = []
    for dj in range(1, N_DEV):
        d = lax.rem(my + dj, N_DEV)
        rdma = pltpu.make_async_remote_copy(
            src_ref=red_ref,
            dst_ref=flat_ref.at[pl.ds(my * CHUNK, CHUNK), :],
            send_sem=s2_send.at[dj - 1],
            recv_sem=s2_recv.at[dj - 1],
            device_id=(d,),
            device_id_type=_MESH,
        )
        rdma.start()
        sends2.append(rdma)

    for k in range(N_DEV - 1):
        recv = pltpu.make_async_remote_copy(
            src_ref=red_ref, dst_ref=red_ref,
            send_sem=s2_send.at[k], recv_sem=s2_recv.at[k],
            device_id=(my,), device_id_type=_MESH,
        )
        recv.wait_recv()
    for r in sends2:
        r.wait_send()

    out_ref[0, :, :] = flat_ref[0:SQ, :].astype(jnp.float32)
    out_ref[1, :, :] = flat_ref[SQ:ROWS, :].astype(jnp.float32)


def kernel(x, Wq, K_ext, V_ext, Wo):
    k_flat = K_ext.reshape(B, SKV, N_DEV, D_LOCAL)
    v_flat = V_ext.reshape(B, SKV, N_DEV, D_LOCAL)

    return pl.pallas_call(
        _body,
        out_shape=jax.ShapeDtypeStruct((B, SQ, D_MODEL), jnp.float32),
        in_specs=[
            pl.BlockSpec(memory_space=pltpu.VMEM),
            pl.BlockSpec(memory_space=pltpu.VMEM),
            pl.BlockSpec(memory_space=pl.ANY),
            pl.BlockSpec(memory_space=pl.ANY),
            pl.BlockSpec(memory_space=pltpu.VMEM),
        ],
        out_specs=pl.BlockSpec(memory_space=pltpu.VMEM),
        scratch_shapes=[
            pltpu.VMEM((B, SKV, D_LOCAL), jnp.float32),
            pltpu.VMEM((B, SKV, D_LOCAL), jnp.float32),
            pltpu.VMEM((ROWS, D_MODEL), jnp.bfloat16),
            pltpu.VMEM((B, SQ, D_LOCAL), jnp.bfloat16),
            pltpu.VMEM((N_DEV - 1, CHUNK, D_MODEL), jnp.bfloat16),
            pltpu.VMEM((CHUNK, D_MODEL), jnp.bfloat16),
            pltpu.VMEM((ROWS, D_MODEL), jnp.bfloat16),
            pltpu.SemaphoreType.DMA((2 * B,)),
            pltpu.SemaphoreType.DMA((N_DEV - 1,)),
            pltpu.SemaphoreType.DMA((N_DEV - 1,)),
            pltpu.SemaphoreType.DMA((N_DEV - 1,)),
            pltpu.SemaphoreType.DMA((N_DEV - 1,)),
        ],
        compiler_params=pltpu.CompilerParams(collective_id=0),
    )(x.astype(jnp.bfloat16), Wq.astype(jnp.bfloat16), k_flat, v_flat,
      Wo.astype(jnp.bfloat16))


# device time: 199838 ns/iter; 1.0008x vs baseline; 1.0008x over previous
import jax
import jax.numpy as jnp
from jax import lax
from jax.experimental import pallas as pl
from jax.experimental.pallas import tpu as pltpu

N_DEV = 16
B, SQ, SKV, DH = 2, 512, 512, 64
HQ_LOCAL = 8
D_LOCAL = HQ_LOCAL * DH
D_MODEL = 768
ROWS = B * SQ
CHUNK = ROWS // N_DEV
NSPLIT = 4
SEG = SKV // NSPLIT

_MESH = pl.DeviceIdType.MESH


def _body(x_ref, wq_ref, k_hbm, v_hbm, wo_ref, out_ref,
          k_scr, v_scr, snd_ref, ctx_ref, a2a_ref, red_ref, flat_ref,
          kv_sems, s1_send, s1_recv, s2_send, s2_recv):
    my = lax.axis_index("i")

    kv_copies = []
    for b in range(B):
        per_b = []
        for s in range(NSPLIT):
            r = pl.ds(s * SEG, SEG)
            ck = pltpu.make_async_copy(
                k_hbm.at[b, r, my, :], k_scr.at[b, r, :],
                kv_sems.at[2 * NSPLIT * b + 2 * s])
            cv = pltpu.make_async_copy(
                v_hbm.at[b, r, my, :], v_scr.at[b, r, :],
                kv_sems.at[2 * NSPLIT * b + 2 * s + 1])
            ck.start()
            cv.start()
            per_b.extend((ck, cv))
        kv_copies.append(per_b)

    barrier_sem = pltpu.get_barrier_semaphore()
    for dj in range(1, N_DEV):
        peer = lax.rem(my + dj, N_DEV)
        pl.semaphore_signal(barrier_sem, inc=1, device_id=(peer,),
                            device_id_type=_MESH)
    pl.semaphore_wait(barrier_sem, N_DEV - 1)

    qb_i = lax.broadcasted_iota(jnp.int32, (SQ, SKV), 0) // 64
    kb_i = lax.broadcasted_iota(jnp.int32, (SQ, SKV), 1) // 64
    bias = jnp.where((kb_i % 4) == (qb_i % 4), 0.0, -1e9)
    for b in range(B):
        q_b = jnp.dot(x_ref[b], wq_ref[...],
                      preferred_element_type=jnp.float32).astype(jnp.bfloat16)
        for c in kv_copies[b]:
            c.wait()
        k_all = k_scr[b].astype(jnp.bfloat16)
        v_all = v_scr[b].astype(jnp.bfloat16)
        for h in range(HQ_LOCAL):
            q = q_b[:, h * DH:(h + 1) * DH]
            k = k_all[:, h * DH:(h + 1) * DH]
            s = lax.dot_general(q, k, (((1,), (1,)), ((), ())),
                                preferred_element_type=jnp.float32)
            w = jnp.exp(s * 0.125 + bias)
            w = w / jnp.sum(w, axis=1, keepdims=True)
            ctx = jnp.dot(w.astype(jnp.bfloat16), v_all[:, h * DH:(h + 1) * DH],
                          preferred_element_type=jnp.float32)
            ctx_ref[b, :, h * DH:(h + 1) * DH] = ctx.astype(jnp.bfloat16)
        proj = jnp.dot(ctx_ref[b], wo_ref[...],
                       preferred_element_type=jnp.float32)
        snd_ref[b * SQ:(b + 1) * SQ, :] = proj.astype(jnp.bfloat16)

    sends1 = []
    for dj in range(1, N_DEV):
        d = lax.rem(my + dj, N_DEV)
        rdma = pltpu.make_async_remote_copy(
            src_ref=snd_ref.at[pl.ds(d * CHUNK, CHUNK), :],
            dst_ref=a2a_ref.at[dj - 1],
            send_sem=s1_send.at[dj - 1],
            recv_sem=s1_recv.at[dj - 1],
            device_id=(d,),
            device_id_type=_MESH,
        )
        rdma.start()
        sends1.append(rdma)

    red = snd_ref[pl.ds(my * CHUNK, CHUNK), :].astype(jnp.float32)
    for k in range(N_DEV - 1):
        recv = pltpu.make_async_remote_copy(
            src_ref=a2a_ref.at[k], dst_ref=a2a_ref.at[k],
            send_sem=s1_send.at[k], recv_sem=s1_recv.at[k],
            device_id=(my,), device_id_type=_MESH,
        )
        recv.wait_recv()
        red = red + a2a_ref[k].astype(jnp.float32)
    red_ref[...] = red.astype(jnp.bfloat16)
    flat_ref[pl.ds(my * CHUNK, CHUNK), :] = red_ref[...]
    for r in sends1:
        r.wait_send()

    sends2 = []
    for dj in range(1, N_DEV):
        d = lax.rem(my + dj, N_DEV)
        rdma = pltpu.make_async_remote_copy(
            src_ref=red_ref,
            dst_ref=flat_ref.at[pl.ds(my * CHUNK, CHUNK), :],
            send_sem=s2_send.at[dj - 1],
            recv_sem=s2_recv.at[dj - 1],
            device_id=(d,),
            device_id_type=_MESH,
        )
        rdma.start()
        sends2.append(rdma)

    for k in range(N_DEV - 1):
        recv = pltpu.make_async_remote_copy(
            src_ref=red_ref, dst_ref=red_ref,
            send_sem=s2_send.at[k], recv_sem=s2_recv.at[k],
            device_id=(my,), device_id_type=_MESH,
        )
        recv.wait_recv()
    for r in sends2:
        r.wait_send()

    out_ref[0, :, :] = flat_ref[0:SQ, :].astype(jnp.float32)
    out_ref[1, :, :] = flat_ref[SQ:ROWS, :].astype(jnp.float32)


def kernel(x, Wq, K_ext, V_ext, Wo):
    k_flat = K_ext.reshape(B, SKV, N_DEV, D_LOCAL)
    v_flat = V_ext.reshape(B, SKV, N_DEV, D_LOCAL)

    return pl.pallas_call(
        _body,
        out_shape=jax.ShapeDtypeStruct((B, SQ, D_MODEL), jnp.float32),
        in_specs=[
            pl.BlockSpec(memory_space=pltpu.VMEM),
            pl.BlockSpec(memory_space=pltpu.VMEM),
            pl.BlockSpec(memory_space=pl.ANY),
            pl.BlockSpec(memory_space=pl.ANY),
            pl.BlockSpec(memory_space=pltpu.VMEM),
        ],
        out_specs=pl.BlockSpec(memory_space=pltpu.VMEM),
        scratch_shapes=[
            pltpu.VMEM((B, SKV, D_LOCAL), jnp.float32),
            pltpu.VMEM((B, SKV, D_LOCAL), jnp.float32),
            pltpu.VMEM((ROWS, D_MODEL), jnp.bfloat16),
            pltpu.VMEM((B, SQ, D_LOCAL), jnp.bfloat16),
            pltpu.VMEM((N_DEV - 1, CHUNK, D_MODEL), jnp.bfloat16),
            pltpu.VMEM((CHUNK, D_MODEL), jnp.bfloat16),
            pltpu.VMEM((ROWS, D_MODEL), jnp.bfloat16),
            pltpu.SemaphoreType.DMA((2 * B * NSPLIT,)),
            pltpu.SemaphoreType.DMA((N_DEV - 1,)),
            pltpu.SemaphoreType.DMA((N_DEV - 1,)),
            pltpu.SemaphoreType.DMA((N_DEV - 1,)),
            pltpu.SemaphoreType.DMA((N_DEV - 1,)),
        ],
        compiler_params=pltpu.CompilerParams(collective_id=0),
    )(x.astype(jnp.bfloat16), Wq.astype(jnp.bfloat16), k_flat, v_flat,
      Wo.astype(jnp.bfloat16))


# device time: 113723 ns/iter; 1.7586x vs baseline; 1.7572x over previous
import jax
import jax.numpy as jnp
from jax import lax
from jax.experimental import pallas as pl
from jax.experimental.pallas import tpu as pltpu

N_DEV = 16
B, SQ, SKV, DH = 2, 512, 512, 64
HQ_LOCAL = 8
D_LOCAL = HQ_LOCAL * DH
D_MODEL = 768
ROWS = B * SQ
CHUNK = ROWS // N_DEV

_MESH = pl.DeviceIdType.MESH


def _body(x_ref, wq_ref, k_ref, v_ref, wo_ref, out_ref,
          snd_ref, ctx_ref, a2a_ref, red_ref, flat_ref,
          s1_send, s1_recv, s2_send, s2_recv):
    my = lax.axis_index("i")

    barrier_sem = pltpu.get_barrier_semaphore()
    for dj in range(1, N_DEV):
        peer = lax.rem(my + dj, N_DEV)
        pl.semaphore_signal(barrier_sem, inc=1, device_id=(peer,),
                            device_id_type=_MESH)
    pl.semaphore_wait(barrier_sem, N_DEV - 1)

    qb_i = lax.broadcasted_iota(jnp.int32, (SQ, SKV), 0) // 64
    kb_i = lax.broadcasted_iota(jnp.int32, (SQ, SKV), 1) // 64
    bias = jnp.where((kb_i % 4) == (qb_i % 4), 0.0, -1e9)
    for b in range(B):
        q_b = jnp.dot(x_ref[b], wq_ref[...],
                      preferred_element_type=jnp.float32).astype(jnp.bfloat16)
        k_all = k_ref[b]
        v_all = v_ref[b]
        for h in range(HQ_LOCAL):
            q = q_b[:, h * DH:(h + 1) * DH]
            k = k_all[:, h * DH:(h + 1) * DH]
            s = lax.dot_general(q, k, (((1,), (1,)), ((), ())),
                                preferred_element_type=jnp.float32)
            w = jnp.exp(s * 0.125 + bias)
            w = w / jnp.sum(w, axis=1, keepdims=True)
            ctx = jnp.dot(w.astype(jnp.bfloat16), v_all[:, h * DH:(h + 1) * DH],
                          preferred_element_type=jnp.float32)
            ctx_ref[b, :, h * DH:(h + 1) * DH] = ctx.astype(jnp.bfloat16)
        proj = jnp.dot(ctx_ref[b], wo_ref[...],
                       preferred_element_type=jnp.float32)
        snd_ref[b * SQ:(b + 1) * SQ, :] = proj.astype(jnp.bfloat16)

    sends1 = []
    for dj in range(1, N_DEV):
        d = lax.rem(my + dj, N_DEV)
        rdma = pltpu.make_async_remote_copy(
            src_ref=snd_ref.at[pl.ds(d * CHUNK, CHUNK), :],
            dst_ref=a2a_ref.at[dj - 1],
            send_sem=s1_send.at[dj - 1],
            recv_sem=s1_recv.at[dj - 1],
            device_id=(d,),
            device_id_type=_MESH,
        )
        rdma.start()
        sends1.append(rdma)

    red = snd_ref[pl.ds(my * CHUNK, CHUNK), :].astype(jnp.float32)
    for k in range(N_DEV - 1):
        recv = pltpu.make_async_remote_copy(
            src_ref=a2a_ref.at[k], dst_ref=a2a_ref.at[k],
            send_sem=s1_send.at[k], recv_sem=s1_recv.at[k],
            device_id=(my,), device_id_type=_MESH,
        )
        recv.wait_recv()
        red = red + a2a_ref[k].astype(jnp.float32)
    red_ref[...] = red.astype(jnp.bfloat16)
    flat_ref[pl.ds(my * CHUNK, CHUNK), :] = red_ref[...]
    for r in sends1:
        r.wait_send()

    sends2 = []
    for dj in range(1, N_DEV):
        d = lax.rem(my + dj, N_DEV)
        rdma = pltpu.make_async_remote_copy(
            src_ref=red_ref,
            dst_ref=flat_ref.at[pl.ds(my * CHUNK, CHUNK), :],
            send_sem=s2_send.at[dj - 1],
            recv_sem=s2_recv.at[dj - 1],
            device_id=(d,),
            device_id_type=_MESH,
        )
        rdma.start()
        sends2.append(rdma)

    for k in range(N_DEV - 1):
        recv = pltpu.make_async_remote_copy(
            src_ref=red_ref, dst_ref=red_ref,
            send_sem=s2_send.at[k], recv_sem=s2_recv.at[k],
            device_id=(my,), device_id_type=_MESH,
        )
        recv.wait_recv()
    for r in sends2:
        r.wait_send()

    out_ref[0, :, :] = flat_ref[0:SQ, :].astype(jnp.float32)
    out_ref[1, :, :] = flat_ref[SQ:ROWS, :].astype(jnp.float32)


def kernel(x, Wq, K_ext, V_ext, Wo):
    i = lax.axis_index("i")
    k_sl = lax.dynamic_slice_in_dim(K_ext, i * HQ_LOCAL, HQ_LOCAL, axis=2)
    v_sl = lax.dynamic_slice_in_dim(V_ext, i * HQ_LOCAL, HQ_LOCAL, axis=2)
    k_sl = k_sl.astype(jnp.bfloat16).reshape(B, SKV, D_LOCAL)
    v_sl = v_sl.astype(jnp.bfloat16).reshape(B, SKV, D_LOCAL)

    return pl.pallas_call(
        _body,
        out_shape=jax.ShapeDtypeStruct((B, SQ, D_MODEL), jnp.float32),
        in_specs=[pl.BlockSpec(memory_space=pltpu.VMEM)] * 5,
        out_specs=pl.BlockSpec(memory_space=pltpu.VMEM),
        scratch_shapes=[
            pltpu.VMEM((ROWS, D_MODEL), jnp.bfloat16),
            pltpu.VMEM((B, SQ, D_LOCAL), jnp.bfloat16),
            pltpu.VMEM((N_DEV - 1, CHUNK, D_MODEL), jnp.bfloat16),
            pltpu.VMEM((CHUNK, D_MODEL), jnp.bfloat16),
            pltpu.VMEM((ROWS, D_MODEL), jnp.bfloat16),
            pltpu.SemaphoreType.DMA((N_DEV - 1,)),
            pltpu.SemaphoreType.DMA((N_DEV - 1,)),
            pltpu.SemaphoreType.DMA((N_DEV - 1,)),
            pltpu.SemaphoreType.DMA((N_DEV - 1,)),
        ],
        compiler_params=pltpu.CompilerParams(collective_id=0),
    )(x.astype(jnp.bfloat16), Wq.astype(jnp.bfloat16), k_sl, v_sl,
      Wo.astype(jnp.bfloat16))
